# baseline (speedup 1.0000x reference)
import functools

import jax
import jax.numpy as jnp
from jax import lax
from jax.experimental import pallas as pl
from jax.experimental.pallas import tpu as pltpu
from jax.experimental.pallas import tpu_sc as plsc

_NUM_KEYS = 4
_BATCH = 4096
_MAX_LEN = 200
_NSEG = _NUM_KEYS * _BATCH
_NC, _NS = 2, 16
_NW = _NC * _NS
_SEG_PER_W = _NSEG // _NW
_WCOLS = 224
_BUF = _SEG_PER_W * (_MAX_LEN - 1) + 7 + 208 + 17
_LWIN = 2 * _SEG_PER_W
_BLK = 8192
_UNROLL = 4


@functools.lru_cache(maxsize=None)
def _sc_fill(total: int):
    mesh = plsc.VectorSubcoreMesh(
        core_axis_name="c", subcore_axis_name="s",
        num_cores=_NC, num_subcores=_NS)

    @functools.partial(
        pl.kernel,
        out_type=jax.ShapeDtypeStruct((total,), jnp.float32),
        mesh=mesh,
        compiler_params=pltpu.CompilerParams(needs_layout_passes=False),
        scratch_types=[
            pltpu.VMEM((_LWIN + 16,), jnp.int32),
            pltpu.VMEM((_NUM_KEYS * _WCOLS,), jnp.float32),
            pltpu.VMEM((16,), jnp.float32),
            pltpu.VMEM((16,), jnp.int32),
            pltpu.VMEM((_NW, 16), jnp.int32),
            pltpu.VMEM_SHARED((_NW, 16), jnp.int32),
            pltpu.VMEM((_BUF,), jnp.float32),
            pltpu.SemaphoreType.DMA,
        ],
    )
    def k(len_hbm, w0_hbm, w1_hbm, w2_hbm, w3_hbm, out_hbm,
          len_v, wmat_v, stage_v, sum_v, sums_v, shared_sums,
          buf, wsem):
        cid = lax.axis_index("c")
        sid = lax.axis_index("s")
        w = sid * _NC + cid
        key_w = w // (_NW // _NUM_KEYS)
        seg0 = w * _SEG_PER_W

        w_copies = [
            pltpu.make_async_copy(wr, wmat_v.at[pl.ds(i * _WCOLS, _MAX_LEN)],
                                  wsem)
            for i, wr in enumerate((w0_hbm, w1_hbm, w2_hbm, w3_hbm))
        ]
        for cp in w_copies:
            cp.start()
        pltpu.sync_copy(len_hbm.at[pl.ds(sid * _LWIN, _LWIN)],
                        len_v.at[pl.ds(0, _LWIN)])

        def vsum(lo_vec, n_vec):
            def body(t, acc):
                return acc + len_v[pl.ds((lo_vec + t) * 16, 16)]
            acc = lax.fori_loop(0, n_vec, body, jnp.zeros((16,), jnp.int32))
            s = acc[0]
            for l in range(1, 16):
                s = s + acc[l]
            return s

        for d in range(2):
            c_id = 2 * sid + d
            csum = vsum(d * (_SEG_PER_W // 16), _SEG_PER_W // 16)
            sum_v[pl.ds(0, 16)] = jnp.full((16,), 0, jnp.int32) + csum
            pltpu.sync_copy(sum_v, shared_sums.at[c_id])
        plsc.subcore_barrier()
        pltpu.sync_copy(shared_sums, sums_v)

        span_lo = jnp.int32(0)
        chunk = jnp.int32(0)
        for j in range(_NW):
            sj = sums_v[j, pl.ds(0, 16)][0]
            span_lo = span_lo + jnp.where(j < w, sj, 0)
            chunk = chunk + jnp.where(j == w, sj, 0)
        span_hi = span_lo + chunk
        a_w = (span_lo // 8) * 8
        r = span_lo - a_w

        mid_lo = ((span_lo + 7) // 8) * 8
        mid_hi = jnp.maximum(mid_lo, (span_hi // 8) * 8)
        head_end = jnp.minimum(mid_lo, span_hi)
        k_h = head_end - span_lo
        tail_lo = jnp.maximum(mid_hi, head_end)
        k_t = span_hi - tail_lo
        msize = mid_hi - mid_lo

        for cp in w_copies:
            cp.wait()
        n_wvec = (_MAX_LEN - 1 + 15) // 16
        wbase = key_w * _WCOLS
        wvecs = [wmat_v[pl.ds(wbase + 16 * j, 16)] for j in range(n_wvec)]
        lbase = _SEG_PER_W * cid
        src0 = mid_lo - a_w
        nb = msize // _BLK

        def fire(blk, c):
            o = blk * _BLK
            pltpu.make_async_copy(buf.at[pl.ds(src0 + o, _BLK)],
                                  out_hbm.at[pl.ds(mid_lo + o, _BLK)],
                                  wsem).start()
            return c

        def seg_block(ib, carry):
            pos, nf = carry
            lv = len_v[pl.ds(lbase + ib * _UNROLL, 16)]
            for l in range(_UNROLL):
                for j in range(n_wvec):
                    buf[pl.ds(pos + 16 * j, 16)] = wvecs[j]
                pos = pos + lv[l]
            tgt = jnp.clip((pos - src0) // _BLK, 0, nb)
            lax.fori_loop(nf, tgt, fire, 0)
            return pos, tgt

        _, nfired = lax.fori_loop(0, _SEG_PER_W // _UNROLL, seg_block,
                                  (r, jnp.int32(0)))

        lax.fori_loop(nfired, nb, fire, 0)
        bit = _BLK // 2
        while bit >= 8:
            o = (msize // (2 * bit)) * (2 * bit)

            @pl.when((msize & bit) != 0)
            def _(o=o, bit=bit):
                pltpu.make_async_copy(buf.at[pl.ds(src0 + o, bit)],
                                      out_hbm.at[pl.ds(mid_lo + o, bit)],
                                      wsem).start()

            bit //= 2

        iota16 = lax.iota(jnp.int32, 16)

        def edge_scatter(lo, cnt, src_base):
            @pl.when(cnt > 0)
            def _():
                lane = jnp.minimum(iota16, cnt - 1)
                src_idx = jnp.maximum(src_base, 0) + lane
                stage_v[pl.ds(0, 16)] = plsc.load_gather(buf, [src_idx])
                dst_idx = lo + lane
                pltpu.sync_copy(stage_v, out_hbm.at[dst_idx])

        edge_scatter(span_lo, k_h, r)
        edge_scatter(tail_lo, k_t, tail_lo - a_w)

        def drain(blk, c):
            o = blk * _BLK
            pltpu.make_async_copy(buf.at[pl.ds(src0 + o, _BLK)],
                                  out_hbm.at[pl.ds(mid_lo + o, _BLK)],
                                  wsem).wait()
            return c

        lax.fori_loop(0, nb, drain, 0)
        bit = _BLK // 2
        while bit >= 8:
            o = (msize // (2 * bit)) * (2 * bit)

            @pl.when((msize & bit) != 0)
            def _(o=o, bit=bit):
                pltpu.make_async_copy(buf.at[pl.ds(src0 + o, bit)],
                                      out_hbm.at[pl.ds(mid_lo + o, bit)],
                                      wsem).wait()

            bit //= 2

    return k


def kernel(values, lengths, W0, W1, W2, W3):
    total = values.shape[0]
    if total == 0:
        return jnp.zeros((0,), jnp.float32)
    return _sc_fill(total)(lengths.astype(jnp.int32), W0, W1, W2, W3)


# speedup vs baseline: 1.0470x; 1.0470x over previous
import functools

import jax
import jax.numpy as jnp
from jax import lax
from jax.experimental import pallas as pl
from jax.experimental.pallas import tpu as pltpu
from jax.experimental.pallas import tpu_sc as plsc

_NUM_KEYS = 4
_BATCH = 4096
_MAX_LEN = 200
_NSEG = _NUM_KEYS * _BATCH
_NC, _NS = 2, 16
_NW = _NC * _NS
_SEG_PER_W = _NSEG // _NW
_WCOLS = 224
_BUF = 544 * (_MAX_LEN - 1) + 7 + 208 + 17
_LWIN = 2 * _SEG_PER_W
_BLK = 8192
_UNROLL = 8
_N_C0, _N_C1 = 480, 544


@functools.lru_cache(maxsize=None)
def _sc_fill(total: int):
    mesh = plsc.VectorSubcoreMesh(
        core_axis_name="c", subcore_axis_name="s",
        num_cores=_NC, num_subcores=_NS)

    @functools.partial(
        pl.kernel,
        out_type=jax.ShapeDtypeStruct((total,), jnp.float32),
        mesh=mesh,
        compiler_params=pltpu.CompilerParams(needs_layout_passes=False),
        scratch_types=[
            pltpu.VMEM((_LWIN + 16,), jnp.int32),
            pltpu.VMEM((_NUM_KEYS * _WCOLS,), jnp.float32),
            pltpu.VMEM((16,), jnp.float32),
            pltpu.VMEM((16,), jnp.int32),
            pltpu.VMEM((_NW, 16), jnp.int32),
            pltpu.VMEM_SHARED((_NW, 16), jnp.int32),
            pltpu.VMEM((_BUF,), jnp.float32),
            pltpu.SemaphoreType.DMA,
        ],
    )
    def k(len_hbm, w0_hbm, w1_hbm, w2_hbm, w3_hbm, out_hbm,
          len_v, wmat_v, stage_v, sum_v, sums_v, shared_sums,
          buf, wsem):
        cid = lax.axis_index("c")
        sid = lax.axis_index("s")
        w = sid * _NC + cid
        key_w = w // (_NW // _NUM_KEYS)

        w_copies = [
            pltpu.make_async_copy(wr, wmat_v.at[pl.ds(i * _WCOLS, _MAX_LEN)],
                                  wsem)
            for i, wr in enumerate((w0_hbm, w1_hbm, w2_hbm, w3_hbm))
        ]
        for cp in w_copies:
            cp.start()
        pltpu.sync_copy(len_hbm.at[pl.ds(sid * _LWIN, _LWIN)],
                        len_v.at[pl.ds(0, _LWIN)])

        def vsum(lo_vec, n_vec):
            def body(t, acc):
                return acc + len_v[pl.ds((lo_vec + t) * 16, 16)]
            acc = lax.fori_loop(0, n_vec, body, jnp.zeros((16,), jnp.int32))
            s = acc[0]
            for l in range(1, 16):
                s = s + acc[l]
            return s

        for d, (lo, n) in enumerate(((0, _N_C0), (_N_C0, _N_C1))):
            c_id = 2 * sid + d
            csum = vsum(lo // 16, n // 16)
            sum_v[pl.ds(0, 16)] = jnp.full((16,), 0, jnp.int32) + csum
            pltpu.sync_copy(sum_v, shared_sums.at[c_id])
        plsc.subcore_barrier()
        pltpu.sync_copy(shared_sums, sums_v)

        span_lo = jnp.int32(0)
        chunk = jnp.int32(0)
        for j in range(_NW):
            sj = sums_v[j, pl.ds(0, 16)][0]
            span_lo = span_lo + jnp.where(j < w, sj, 0)
            chunk = chunk + jnp.where(j == w, sj, 0)
        span_hi = span_lo + chunk
        a_w = (span_lo // 8) * 8
        r = span_lo - a_w

        mid_lo = ((span_lo + 7) // 8) * 8
        mid_hi = jnp.maximum(mid_lo, (span_hi // 8) * 8)
        head_end = jnp.minimum(mid_lo, span_hi)
        k_h = head_end - span_lo
        tail_lo = jnp.maximum(mid_hi, head_end)
        k_t = span_hi - tail_lo
        msize = mid_hi - mid_lo

        for cp in w_copies:
            cp.wait()
        n_wvec = (_MAX_LEN - 1 + 15) // 16
        wbase = key_w * _WCOLS
        wvecs = [wmat_v[pl.ds(wbase + 16 * j, 16)] for j in range(n_wvec)]
        lbase = cid * _N_C0
        ntrip = jnp.where(cid == 0, _N_C0 // _UNROLL, _N_C1 // _UNROLL)
        src0 = mid_lo - a_w
        nb = msize // _BLK

        def fire(blk, c):
            o = blk * _BLK
            pltpu.make_async_copy(buf.at[pl.ds(src0 + o, _BLK)],
                                  out_hbm.at[pl.ds(mid_lo + o, _BLK)],
                                  wsem).start()
            return c

        def seg_block(ib, carry):
            pos, nf = carry
            lv = len_v[pl.ds(lbase + ib * _UNROLL, 16)]
            for l in range(_UNROLL):
                for j in range(n_wvec):
                    buf[pl.ds(pos + 16 * j, 16)] = wvecs[j]
                pos = pos + lv[l]
            tgt = jnp.clip((pos - src0) // _BLK, 0, nb)
            lax.fori_loop(nf, tgt, fire, 0)
            return pos, tgt

        _, nfired = lax.fori_loop(0, ntrip, seg_block, (r, jnp.int32(0)))

        lax.fori_loop(nfired, nb, fire, 0)
        bit = _BLK // 2
        while bit >= 8:
            o = (msize // (2 * bit)) * (2 * bit)

            @pl.when((msize & bit) != 0)
            def _(o=o, bit=bit):
                pltpu.make_async_copy(buf.at[pl.ds(src0 + o, bit)],
                                      out_hbm.at[pl.ds(mid_lo + o, bit)],
                                      wsem).start()

            bit //= 2

        iota16 = lax.iota(jnp.int32, 16)

        def edge_scatter(lo, cnt, src_base):
            @pl.when(cnt > 0)
            def _():
                lane = jnp.minimum(iota16, cnt - 1)
                src_idx = jnp.maximum(src_base, 0) + lane
                stage_v[pl.ds(0, 16)] = plsc.load_gather(buf, [src_idx])
                dst_idx = lo + lane
                pltpu.sync_copy(stage_v, out_hbm.at[dst_idx])

        edge_scatter(span_lo, k_h, r)
        edge_scatter(tail_lo, k_t, tail_lo - a_w)

        def drain(blk, c):
            o = blk * _BLK
            pltpu.make_async_copy(buf.at[pl.ds(src0 + o, _BLK)],
                                  out_hbm.at[pl.ds(mid_lo + o, _BLK)],
                                  wsem).wait()
            return c

        lax.fori_loop(0, nb, drain, 0)
        bit = _BLK // 2
        while bit >= 8:
            o = (msize // (2 * bit)) * (2 * bit)

            @pl.when((msize & bit) != 0)
            def _(o=o, bit=bit):
                pltpu.make_async_copy(buf.at[pl.ds(src0 + o, bit)],
                                      out_hbm.at[pl.ds(mid_lo + o, bit)],
                                      wsem).wait()

            bit //= 2

    return k


def kernel(values, lengths, W0, W1, W2, W3):
    total = values.shape[0]
    if total == 0:
        return jnp.zeros((0,), jnp.float32)
    return _sc_fill(total)(lengths.astype(jnp.int32), W0, W1, W2, W3)


# speedup vs baseline: 1.0493x; 1.0022x over previous
import functools

import jax
import jax.numpy as jnp
from jax import lax
from jax.experimental import pallas as pl
from jax.experimental.pallas import tpu as pltpu
from jax.experimental.pallas import tpu_sc as plsc

_NUM_KEYS = 4
_BATCH = 4096
_MAX_LEN = 200
_NSEG = _NUM_KEYS * _BATCH
_NC, _NS = 2, 16
_NW = _NC * _NS
_SEG_PER_W = _NSEG // _NW
_WCOLS = 224
_BUF = 544 * (_MAX_LEN - 1) + 7 + 208 + 17
_LWIN = 2 * _SEG_PER_W
_BLK = 8192
_UNROLL = 16
_N_C0, _N_C1 = 480, 544


@functools.lru_cache(maxsize=None)
def _sc_fill(total: int):
    mesh = plsc.VectorSubcoreMesh(
        core_axis_name="c", subcore_axis_name="s",
        num_cores=_NC, num_subcores=_NS)

    @functools.partial(
        pl.kernel,
        out_type=jax.ShapeDtypeStruct((total,), jnp.float32),
        mesh=mesh,
        compiler_params=pltpu.CompilerParams(needs_layout_passes=False),
        scratch_types=[
            pltpu.VMEM((_LWIN + 16,), jnp.int32),
            pltpu.VMEM((_NUM_KEYS * _WCOLS,), jnp.float32),
            pltpu.VMEM((16,), jnp.float32),
            pltpu.VMEM((16,), jnp.int32),
            pltpu.VMEM((_NW, 16), jnp.int32),
            pltpu.VMEM_SHARED((_NW, 16), jnp.int32),
            pltpu.VMEM((_BUF,), jnp.float32),
            pltpu.SemaphoreType.DMA,
        ],
    )
    def k(len_hbm, w0_hbm, w1_hbm, w2_hbm, w3_hbm, out_hbm,
          len_v, wmat_v, stage_v, sum_v, sums_v, shared_sums,
          buf, wsem):
        cid = lax.axis_index("c")
        sid = lax.axis_index("s")
        w = sid * _NC + cid
        key_w = w // (_NW // _NUM_KEYS)

        w_copies = [
            pltpu.make_async_copy(wr, wmat_v.at[pl.ds(i * _WCOLS, _MAX_LEN)],
                                  wsem)
            for i, wr in enumerate((w0_hbm, w1_hbm, w2_hbm, w3_hbm))
        ]
        for cp in w_copies:
            cp.start()
        pltpu.sync_copy(len_hbm.at[pl.ds(sid * _LWIN, _LWIN)],
                        len_v.at[pl.ds(0, _LWIN)])

        def vsum(lo_vec, n_vec):
            def body(t, acc):
                return acc + len_v[pl.ds((lo_vec + t) * 16, 16)]
            acc = lax.fori_loop(0, n_vec, body, jnp.zeros((16,), jnp.int32))
            s = acc[0]
            for l in range(1, 16):
                s = s + acc[l]
            return s

        for d, (lo, n) in enumerate(((0, _N_C0), (_N_C0, _N_C1))):
            c_id = 2 * sid + d
            csum = vsum(lo // 16, n // 16)
            sum_v[pl.ds(0, 16)] = jnp.full((16,), 0, jnp.int32) + csum
            pltpu.sync_copy(sum_v, shared_sums.at[c_id])
        plsc.subcore_barrier()
        pltpu.sync_copy(shared_sums, sums_v)

        span_lo = jnp.int32(0)
        chunk = jnp.int32(0)
        for j in range(_NW):
            sj = sums_v[j, pl.ds(0, 16)][0]
            span_lo = span_lo + jnp.where(j < w, sj, 0)
            chunk = chunk + jnp.where(j == w, sj, 0)
        span_hi = span_lo + chunk
        a_w = (span_lo // 8) * 8
        r = span_lo - a_w

        mid_lo = ((span_lo + 7) // 8) * 8
        mid_hi = jnp.maximum(mid_lo, (span_hi // 8) * 8)
        head_end = jnp.minimum(mid_lo, span_hi)
        k_h = head_end - span_lo
        tail_lo = jnp.maximum(mid_hi, head_end)
        k_t = span_hi - tail_lo
        msize = mid_hi - mid_lo

        for cp in w_copies:
            cp.wait()
        n_wvec = (_MAX_LEN - 1 + 15) // 16
        wbase = key_w * _WCOLS
        wvecs = [wmat_v[pl.ds(wbase + 16 * j, 16)] for j in range(n_wvec)]
        lbase = cid * _N_C0
        ntrip = jnp.where(cid == 0, _N_C0 // _UNROLL, _N_C1 // _UNROLL)
        src0 = mid_lo - a_w
        nb = msize // _BLK

        def fire(blk, c):
            o = blk * _BLK
            pltpu.make_async_copy(buf.at[pl.ds(src0 + o, _BLK)],
                                  out_hbm.at[pl.ds(mid_lo + o, _BLK)],
                                  wsem).start()
            return c

        def seg_block(ib, carry):
            pos, nf = carry
            lv = len_v[pl.ds(lbase + ib * _UNROLL, 16)]
            for l in range(_UNROLL):
                for j in range(n_wvec):
                    buf[pl.ds(pos + 16 * j, 16)] = wvecs[j]
                pos = pos + lv[l]
            tgt = jnp.clip((pos - src0) // _BLK, 0, nb)
            lax.fori_loop(nf, tgt, fire, 0)
            return pos, tgt

        _, nfired = lax.fori_loop(0, ntrip, seg_block, (r, jnp.int32(0)))

        lax.fori_loop(nfired, nb, fire, 0)
        bit = _BLK // 2
        while bit >= 8:
            o = (msize // (2 * bit)) * (2 * bit)

            @pl.when((msize & bit) != 0)
            def _(o=o, bit=bit):
                pltpu.make_async_copy(buf.at[pl.ds(src0 + o, bit)],
                                      out_hbm.at[pl.ds(mid_lo + o, bit)],
                                      wsem).start()

            bit //= 2

        iota16 = lax.iota(jnp.int32, 16)

        def edge_scatter(lo, cnt, src_base):
            @pl.when(cnt > 0)
            def _():
                lane = jnp.minimum(iota16, cnt - 1)
                src_idx = jnp.maximum(src_base, 0) + lane
                stage_v[pl.ds(0, 16)] = plsc.load_gather(buf, [src_idx])
                dst_idx = lo + lane
                pltpu.sync_copy(stage_v, out_hbm.at[dst_idx])

        edge_scatter(span_lo, k_h, r)
        edge_scatter(tail_lo, k_t, tail_lo - a_w)

        def drain(blk, c):
            o = blk * _BLK
            pltpu.make_async_copy(buf.at[pl.ds(src0 + o, _BLK)],
                                  out_hbm.at[pl.ds(mid_lo + o, _BLK)],
                                  wsem).wait()
            return c

        lax.fori_loop(0, nb, drain, 0)
        bit = _BLK // 2
        while bit >= 8:
            o = (msize // (2 * bit)) * (2 * bit)

            @pl.when((msize & bit) != 0)
            def _(o=o, bit=bit):
                pltpu.make_async_copy(buf.at[pl.ds(src0 + o, bit)],
                                      out_hbm.at[pl.ds(mid_lo + o, bit)],
                                      wsem).wait()

            bit //= 2

    return k


def kernel(values, lengths, W0, W1, W2, W3):
    total = values.shape[0]
    if total == 0:
        return jnp.zeros((0,), jnp.float32)
    return _sc_fill(total)(lengths.astype(jnp.int32), W0, W1, W2, W3)
